# Initial kernel scaffold; baseline (speedup 1.0000x reference)
import jax
import jax.numpy as jnp
from jax.experimental import pallas as pl


def kernel(log_score, sigma_bar, xt, x0):
    raise NotImplementedError("write your pallas kernel here")


# speedup vs baseline: 2.1259x; 2.1259x over previous
import functools

import jax
import jax.numpy as jnp
from jax import lax
from jax.experimental import pallas as pl
from jax.experimental.pallas import tpu as pltpu
from jax.experimental.pallas import tpu_sc as plsc

NUM_VOCABS = 1024
B, L, V = 8, 4096, 1024
N = B * L
MASK_TOK = NUM_VOCABS - 1

LANES = 16
NC, NS = 2, 16
NW = NC * NS
CHUNK = N // NW
NGROUPS = CHUNK // LANES
VG = V // LANES


def _sc_loss(ls2d, xt_flat, x0_flat, ratio_pad, const_pad):
    mesh = plsc.VectorSubcoreMesh(core_axis_name="c", subcore_axis_name="s")

    @functools.partial(
        pl.kernel,
        mesh=mesh,
        out_type=jax.ShapeDtypeStruct((NW, LANES), jnp.float32),
        compiler_params=pltpu.CompilerParams(needs_layout_passes=False),
        scratch_types=[
            pltpu.VMEM((CHUNK + LANES,), jnp.int32),
            pltpu.VMEM((CHUNK + LANES,), jnp.int32),
            pltpu.VMEM((LANES,), jnp.float32),
            pltpu.VMEM((LANES,), jnp.float32),
            pltpu.VMEM((V + LANES,), jnp.float32),
            pltpu.VMEM((LANES,), jnp.float32),
        ],
    )
    def k(ls_hbm, xt_hbm, x0_hbm, ratio_hbm, const_hbm, out_hbm,
          xt_v, x0_v, ratio_v, const_v, row_v, acc_v):
        wid = lax.axis_index("s") * NC + lax.axis_index("c")
        base = wid * CHUNK
        b = base // L
        pltpu.sync_copy(xt_hbm.at[pl.ds(base, CHUNK)], xt_v.at[pl.ds(0, CHUNK)])
        pltpu.sync_copy(x0_hbm.at[pl.ds(base, CHUNK)], x0_v.at[pl.ds(0, CHUNK)])
        xt_v[pl.ds(CHUNK, LANES)] = jnp.zeros((LANES,), jnp.int32)
        x0_v[pl.ds(CHUNK, LANES)] = jnp.zeros((LANES,), jnp.int32)
        pltpu.sync_copy(ratio_hbm, ratio_v)
        pltpu.sync_copy(const_hbm, const_v)

        lanes = lax.broadcasted_iota(jnp.int32, (LANES,), 0)
        zero16 = jnp.zeros((LANES,), jnp.float32)
        acc_v[...] = zero16
        row_v[pl.ds(V, LANES)] = zero16
        ratio_all = ratio_v[...]
        const_all = const_v[...]
        lane_is_b = lanes == b
        last_lane = lanes == (LANES - 1)

        def group_body(g, carry):
            xt16 = xt_v[pl.ds(g * LANES, LANES)]
            cnt = jnp.sum(jnp.where(xt16 == MASK_TOK, 1, 0))

            @pl.when(cnt > 0)
            def _():
                def row_body(r, rc):
                    p = g * LANES + r
                    tok = xt_v[pl.ds(p, LANES)][0]

                    @pl.when(tok == MASK_TOK)
                    def _():
                        pltpu.sync_copy(ls_hbm.at[base + p],
                                        row_v.at[pl.ds(0, V)])
                        x0r = x0_v[pl.ds(p, LANES)][0]
                        negv = row_v[pl.ds(x0r, LANES)][0]
                        negs = jnp.full((LANES,), negv, jnp.float32)

                        def col_body(j, a):
                            e = jnp.exp(row_v[pl.ds(j * LANES, LANES)])
                            drop = (j == VG - 1) & last_lane
                            return a + jnp.where(drop, 0.0, e)

                        pos_vec = lax.fori_loop(0, VG, col_body, zero16)
                        corr = jnp.where(
                            lane_is_b, const_all - ratio_all * negs, 0.0)
                        acc_v[...] = acc_v[...] + pos_vec + corr
                    return rc

                lax.fori_loop(0, LANES, row_body, 0)
            return carry

        lax.fori_loop(0, NGROUPS, group_body, 0)
        pltpu.sync_copy(acc_v, out_hbm.at[wid])

    return k(ls2d, xt_flat, x0_flat, ratio_pad, const_pad)


def kernel(log_score, sigma_bar, xt, x0):
    expm1_sb = jnp.where(sigma_bar < 0.5, jnp.expm1(sigma_bar),
                         jnp.exp(sigma_bar) - 1.0)
    ratio = 1.0 / expm1_sb
    const = ratio * (jnp.log(ratio) - 1.0)
    ratio_pad = jnp.zeros((LANES,), jnp.float32).at[:B].set(ratio)
    const_pad = jnp.zeros((LANES,), jnp.float32).at[:B].set(const)
    partials = _sc_loss(log_score.reshape(N, V), xt.reshape(N), x0.reshape(N),
                        ratio_pad, const_pad)
    return partials.sum()


# speedup vs baseline: 2.3831x; 1.1210x over previous
import functools

import jax
import jax.numpy as jnp
from jax import lax
from jax.experimental import pallas as pl
from jax.experimental.pallas import tpu as pltpu
from jax.experimental.pallas import tpu_sc as plsc

NUM_VOCABS = 1024
B, L, V = 8, 4096, 1024
N = B * L
MASK_TOK = NUM_VOCABS - 1

LANES = 16
NC, NS = 2, 16
NW = NC * NS
CHUNK = N // NW
NGROUPS = CHUNK // LANES
VG = V // LANES
UNROLL = 4


def _sc_partials(ls2d, xt_flat, x0_flat):
    mesh = plsc.VectorSubcoreMesh(core_axis_name="c", subcore_axis_name="s")

    @functools.partial(
        pl.kernel,
        mesh=mesh,
        out_type=jax.ShapeDtypeStruct((NW, 3, LANES), jnp.float32),
        compiler_params=pltpu.CompilerParams(needs_layout_passes=False),
        scratch_types=[
            pltpu.VMEM((CHUNK + LANES,), jnp.int32),
            pltpu.VMEM((CHUNK + LANES,), jnp.int32),
            pltpu.VMEM((CHUNK + LANES,), jnp.int32),
            pltpu.VMEM((V + LANES,), jnp.float32),
            pltpu.VMEM((3, LANES), jnp.float32),
            pltpu.SemaphoreType.DMA,
            pltpu.SemaphoreType.DMA,
        ],
    )
    def k(ls_hbm, xt_hbm, x0_hbm, out_hbm,
          xt_v, x0_v, idx_v, row_v, acc_v, sem0, sem1):
        wid = lax.axis_index("s") * NC + lax.axis_index("c")
        base = wid * CHUNK
        cp_xt = pltpu.async_copy(
            xt_hbm.at[pl.ds(base, CHUNK)], xt_v.at[pl.ds(0, CHUNK)], sem0)
        cp_x0 = pltpu.async_copy(
            x0_hbm.at[pl.ds(base, CHUNK)], x0_v.at[pl.ds(0, CHUNK)], sem1)

        lanes = lax.broadcasted_iota(jnp.int32, (LANES,), 0)
        zero16 = jnp.zeros((LANES,), jnp.float32)
        acc_v[0, :] = zero16
        acc_v[1, :] = zero16
        acc_v[2, :] = zero16
        row_v[pl.ds(V, LANES)] = zero16
        last_lane = lanes == (LANES - 1)
        cp_xt.wait()
        cp_x0.wait()

        def scan_body(g, cnt):
            p0 = g * LANES
            m = xt_v[pl.ds(p0, LANES)] == MASK_TOK
            plsc.store_compressed(
                idx_v.at[pl.ds(cnt, LANES)], p0 + lanes, mask=m)
            return cnt + plsc.all_reduce_population_count(m)[0]

        nmask = lax.fori_loop(0, NGROUPS, scan_body, 0)

        def row_body(i, _):
            p = idx_v[pl.ds(i, LANES)][0]
            pltpu.sync_copy(ls_hbm.at[base + p], row_v.at[pl.ds(0, V)])
            x0r = x0_v[pl.ds(p, LANES)][0]
            negv = row_v[pl.ds(x0r, LANES)][0]

            def col_body(j, accs):
                c = j * (LANES * UNROLL)
                return tuple(
                    accs[u] + jnp.exp(row_v[pl.ds(c + u * LANES, LANES)])
                    for u in range(UNROLL))

            accs = lax.fori_loop(0, VG // UNROLL, col_body, (zero16,) * UNROLL)
            pos = (accs[0] + accs[1]) + (accs[2] + accs[3])
            pos = pos - jnp.where(
                last_lane, jnp.exp(row_v[pl.ds(V - LANES, LANES)]), 0.0)
            acc_v[0, :] = acc_v[0, :] + pos
            acc_v[1, :] = acc_v[1, :] + jnp.full((LANES,), negv, jnp.float32)
            acc_v[2, :] = acc_v[2, :] + 1.0
            return 0

        lax.fori_loop(0, nmask, row_body, 0)
        pltpu.sync_copy(acc_v, out_hbm.at[wid])

    return k(ls2d, xt_flat, x0_flat)


def kernel(log_score, sigma_bar, xt, x0):
    parts = _sc_partials(log_score.reshape(N, V), xt.reshape(N), x0.reshape(N))
    expm1_sb = jnp.where(sigma_bar < 0.5, jnp.expm1(sigma_bar),
                         jnp.exp(sigma_bar) - 1.0)
    ratio = 1.0 / expm1_sb
    const = ratio * (jnp.log(ratio) - 1.0)
    ratio_w = jnp.repeat(ratio, NW // B)
    const_w = jnp.repeat(const, NW // B)
    pos_sum = parts[:, 0, :].sum()
    neg_w = parts[:, 1, :].sum(axis=1) * (1.0 / LANES)
    cnt_w = parts[:, 2, :].sum(axis=1) * (1.0 / LANES)
    return pos_sum + (const_w * cnt_w - ratio_w * neg_w).sum()


# speedup vs baseline: 2.4352x; 1.0219x over previous
import functools

import jax
import jax.numpy as jnp
from jax import lax
from jax.experimental import pallas as pl
from jax.experimental.pallas import tpu as pltpu
from jax.experimental.pallas import tpu_sc as plsc

NUM_VOCABS = 1024
B, L, V = 8, 4096, 1024
N = B * L
MASK_TOK = NUM_VOCABS - 1

LANES = 16
NC, NS = 2, 16
NW = NC * NS
CHUNK = N // NW
NGROUPS = CHUNK // LANES
VG = V // LANES
UNROLL = 4


def _sc_partials(ls3d, xt2d, x02d):
    mesh = plsc.VectorSubcoreMesh(core_axis_name="c", subcore_axis_name="s")

    @functools.partial(
        pl.kernel,
        mesh=mesh,
        out_type=jax.ShapeDtypeStruct((NW, 3, LANES), jnp.float32),
        compiler_params=pltpu.CompilerParams(
            needs_layout_passes=False, skip_device_barrier=True),
        scratch_types=[
            pltpu.VMEM((CHUNK + LANES,), jnp.int32),
            pltpu.VMEM((CHUNK + LANES,), jnp.int32),
            pltpu.VMEM((CHUNK + LANES,), jnp.int32),
            pltpu.VMEM((V + LANES,), jnp.float32),
            pltpu.VMEM((3, LANES), jnp.float32),
            pltpu.SemaphoreType.DMA,
            pltpu.SemaphoreType.DMA,
        ],
    )
    def k(ls_hbm, xt_hbm, x0_hbm, out_hbm,
          xt_v, x0_v, idx_v, row_v, acc_v, sem0, sem1):
        wid = lax.axis_index("s") * NC + lax.axis_index("c")
        bi = wid // (L // CHUNK)
        loff = (wid % (L // CHUNK)) * CHUNK
        cp_xt = pltpu.async_copy(
            xt_hbm.at[bi, pl.ds(loff, CHUNK)], xt_v.at[pl.ds(0, CHUNK)], sem0)
        cp_x0 = pltpu.async_copy(
            x0_hbm.at[bi, pl.ds(loff, CHUNK)], x0_v.at[pl.ds(0, CHUNK)], sem1)

        lanes = lax.broadcasted_iota(jnp.int32, (LANES,), 0)
        zero16 = jnp.zeros((LANES,), jnp.float32)
        acc_v[0, :] = zero16
        acc_v[1, :] = zero16
        acc_v[2, :] = zero16
        row_v[pl.ds(V, LANES)] = zero16
        last_lane = lanes == (LANES - 1)
        cp_xt.wait()
        cp_x0.wait()

        def scan_body(g, cnt):
            p0 = g * LANES
            m = xt_v[pl.ds(p0, LANES)] == MASK_TOK
            plsc.store_compressed(
                idx_v.at[pl.ds(cnt, LANES)], p0 + lanes, mask=m)
            return cnt + plsc.all_reduce_population_count(m)[0]

        nmask = lax.fori_loop(0, NGROUPS, scan_body, 0)

        def row_body(i, _):
            p = idx_v[pl.ds(i, LANES)][0]
            pltpu.sync_copy(ls_hbm.at[bi, loff + p], row_v.at[pl.ds(0, V)])
            x0r = x0_v[pl.ds(p, LANES)][0]
            negv = row_v[pl.ds(x0r, LANES)][0]

            def col_body(j, accs):
                c = j * (LANES * UNROLL)
                return tuple(
                    accs[u] + jnp.exp(row_v[pl.ds(c + u * LANES, LANES)])
                    for u in range(UNROLL))

            accs = lax.fori_loop(0, VG // UNROLL, col_body, (zero16,) * UNROLL)
            pos = (accs[0] + accs[1]) + (accs[2] + accs[3])
            pos = pos - jnp.where(
                last_lane, jnp.exp(row_v[pl.ds(V - LANES, LANES)]), 0.0)
            acc_v[0, :] = acc_v[0, :] + pos
            acc_v[1, :] = acc_v[1, :] + jnp.full((LANES,), negv, jnp.float32)
            acc_v[2, :] = acc_v[2, :] + 1.0
            return 0

        lax.fori_loop(0, nmask, row_body, 0)
        pltpu.sync_copy(acc_v, out_hbm.at[wid])

    return k(ls3d, xt2d, x02d)


def kernel(log_score, sigma_bar, xt, x0):
    parts = _sc_partials(log_score, xt, x0)
    expm1_sb = jnp.where(sigma_bar < 0.5, jnp.expm1(sigma_bar),
                         jnp.exp(sigma_bar) - 1.0)
    ratio = 1.0 / expm1_sb
    const = ratio * (jnp.log(ratio) - 1.0)
    ratio_w = jnp.repeat(ratio, NW // B)
    const_w = jnp.repeat(const, NW // B)
    pos_sum = parts[:, 0, :].sum()
    neg_w = parts[:, 1, :].sum(axis=1) * (1.0 / LANES)
    cnt_w = parts[:, 2, :].sum(axis=1) * (1.0 / LANES)
    return pos_sum + (const_w * cnt_w - ratio_w * neg_w).sum()


# speedup vs baseline: 2.6601x; 1.0924x over previous
import functools

import jax
import jax.numpy as jnp
from jax import lax
from jax.experimental import pallas as pl
from jax.experimental.pallas import tpu as pltpu
from jax.experimental.pallas import tpu_sc as plsc

NUM_VOCABS = 1024
B, L, V = 8, 4096, 1024
N = B * L
MASK_TOK = NUM_VOCABS - 1

LANES = 16
NC, NS = 2, 16
NW = NC * NS
CHUNK = N // NW
NGROUPS = CHUNK // LANES
VG = V // LANES
UNROLL = 4
WPB = L // CHUNK


def _sc_partials(ls3d, xt2d, x02d, ratio_pad, const_pad):
    mesh = plsc.VectorSubcoreMesh(core_axis_name="c", subcore_axis_name="s")

    @functools.partial(
        pl.kernel,
        mesh=mesh,
        out_type=jax.ShapeDtypeStruct((NW, LANES), jnp.float32),
        compiler_params=pltpu.CompilerParams(
            needs_layout_passes=False, skip_device_barrier=True),
        scratch_types=[
            pltpu.VMEM((CHUNK + LANES,), jnp.int32),
            pltpu.VMEM((CHUNK + LANES,), jnp.int32),
            pltpu.VMEM((CHUNK + LANES,), jnp.int32),
            pltpu.VMEM((V + LANES,), jnp.float32),
            pltpu.VMEM((LANES,), jnp.float32),
            pltpu.VMEM((LANES,), jnp.float32),
            pltpu.VMEM((LANES,), jnp.float32),
            pltpu.SemaphoreType.DMA,
            pltpu.SemaphoreType.DMA,
        ],
    )
    def k(ls_hbm, xt_hbm, x0_hbm, ratio_hbm, const_hbm, out_hbm,
          xt_v, x0_v, idx_v, row_v, ratio_v, const_v, fin_v, sem0, sem1):
        wid = lax.axis_index("s") * NC + lax.axis_index("c")
        bi = wid // WPB
        loff = (wid % WPB) * CHUNK
        cp_xt = pltpu.async_copy(
            xt_hbm.at[bi, pl.ds(loff, CHUNK)], xt_v.at[pl.ds(0, CHUNK)], sem0)
        cp_x0 = pltpu.async_copy(
            x0_hbm.at[bi, pl.ds(loff, CHUNK)], x0_v.at[pl.ds(0, CHUNK)], sem1)
        pltpu.sync_copy(ratio_hbm, ratio_v)
        pltpu.sync_copy(const_hbm, const_v)

        lanes = lax.broadcasted_iota(jnp.int32, (LANES,), 0)
        zero16 = jnp.zeros((LANES,), jnp.float32)
        row_v[pl.ds(V, LANES)] = zero16
        last_lane = lanes == (LANES - 1)
        lane_is_b = lanes == bi
        cp_xt.wait()
        cp_x0.wait()

        def scan_body(g, cnt):
            p0 = g * LANES
            m = xt_v[pl.ds(p0, LANES)] == MASK_TOK
            plsc.store_compressed(
                idx_v.at[pl.ds(cnt, LANES)], p0 + lanes, mask=m)
            return cnt + plsc.all_reduce_population_count(m)[0]

        nmask = lax.fori_loop(0, NGROUPS, scan_body, 0)

        def row_body(i, carry):
            pos_acc, neg_acc = carry
            p = idx_v[pl.ds(i, LANES)][0]
            pltpu.sync_copy(ls_hbm.at[bi, loff + p], row_v.at[pl.ds(0, V)])
            x0r = x0_v[pl.ds(p, LANES)][0]
            negv = row_v[pl.ds(x0r, LANES)][0]

            def col_body(j, accs):
                c = j * (LANES * UNROLL)
                return tuple(
                    accs[u] + jnp.exp(row_v[pl.ds(c + u * LANES, LANES)])
                    for u in range(UNROLL))

            accs = lax.fori_loop(0, VG // UNROLL, col_body, (zero16,) * UNROLL)
            pos = (accs[0] + accs[1]) + (accs[2] + accs[3])
            pos = pos - jnp.where(
                last_lane, jnp.exp(row_v[pl.ds(V - LANES, LANES)]), 0.0)
            return (pos_acc + pos,
                    neg_acc + jnp.full((LANES,), negv, jnp.float32))

        pos_acc, neg_acc = lax.fori_loop(0, nmask, row_body, (zero16, zero16))
        cntf = jnp.full((LANES,), nmask, jnp.int32).astype(jnp.float32)
        fin_v[...] = pos_acc + jnp.where(
            lane_is_b, const_v[...] * cntf - ratio_v[...] * neg_acc, 0.0)
        pltpu.sync_copy(fin_v, out_hbm.at[wid])

    return k(ls3d, xt2d, x02d, ratio_pad, const_pad)


def kernel(log_score, sigma_bar, xt, x0):
    expm1_sb = jnp.where(sigma_bar < 0.5, jnp.expm1(sigma_bar),
                         jnp.exp(sigma_bar) - 1.0)
    ratio = 1.0 / expm1_sb
    const = ratio * (jnp.log(ratio) - 1.0)
    ratio_pad = jnp.zeros((LANES,), jnp.float32).at[:B].set(ratio)
    const_pad = jnp.zeros((LANES,), jnp.float32).at[:B].set(const)
    parts = _sc_partials(log_score, xt, x0, ratio_pad, const_pad)
    return parts.sum()


# speedup vs baseline: 2.6771x; 1.0064x over previous
import functools

import jax
import jax.numpy as jnp
from jax import lax
from jax.experimental import pallas as pl
from jax.experimental.pallas import tpu as pltpu
from jax.experimental.pallas import tpu_sc as plsc

NUM_VOCABS = 1024
B, L, V = 8, 4096, 1024
N = B * L
MASK_TOK = NUM_VOCABS - 1

LANES = 16
NC, NS = 2, 16
NW = NC * NS
CHUNK = N // NW
NGROUPS = CHUNK // LANES
VG = V // LANES
UNROLL = 4
WPB = L // CHUNK


def _sc_partials(ls3d, xt2d, x02d, ratio_pad, const_pad):
    mesh = plsc.VectorSubcoreMesh(core_axis_name="c", subcore_axis_name="s")

    @functools.partial(
        pl.kernel,
        mesh=mesh,
        out_type=jax.ShapeDtypeStruct((NW, LANES), jnp.float32),
        compiler_params=pltpu.CompilerParams(
            needs_layout_passes=False, skip_device_barrier=True),
        scratch_types=[
            pltpu.VMEM((CHUNK + LANES,), jnp.int32),
            pltpu.VMEM((CHUNK + LANES,), jnp.int32),
            pltpu.VMEM((CHUNK + LANES,), jnp.int32),
            pltpu.VMEM((2, V + LANES), jnp.float32),
            pltpu.VMEM((LANES,), jnp.float32),
            pltpu.VMEM((LANES,), jnp.float32),
            pltpu.VMEM((LANES,), jnp.float32),
            pltpu.SemaphoreType.DMA,
            pltpu.SemaphoreType.DMA,
            pltpu.SemaphoreType.DMA,
            pltpu.SemaphoreType.DMA,
        ],
    )
    def k(ls_hbm, xt_hbm, x0_hbm, ratio_hbm, const_hbm, out_hbm,
          xt_v, x0_v, idx_v, row_v, ratio_v, const_v, fin_v,
          sem0, sem1, semA, semB):
        wid = lax.axis_index("s") * NC + lax.axis_index("c")
        bi = wid // WPB
        loff = (wid % WPB) * CHUNK
        cp_xt = pltpu.async_copy(
            xt_hbm.at[bi, pl.ds(loff, CHUNK)], xt_v.at[pl.ds(0, CHUNK)], sem0)
        cp_x0 = pltpu.async_copy(
            x0_hbm.at[bi, pl.ds(loff, CHUNK)], x0_v.at[pl.ds(0, CHUNK)], sem1)
        cp_ratio = pltpu.async_copy(ratio_hbm, ratio_v, semA)
        cp_const = pltpu.async_copy(const_hbm, const_v, semB)

        lanes = lax.broadcasted_iota(jnp.int32, (LANES,), 0)
        zero16 = jnp.zeros((LANES,), jnp.float32)
        row_v[0, pl.ds(V, LANES)] = zero16
        row_v[1, pl.ds(V, LANES)] = zero16
        last_lane = lanes == (LANES - 1)
        lane_is_b = lanes == bi
        cp_ratio.wait()
        cp_const.wait()
        cp_xt.wait()
        cp_x0.wait()

        def scan_body(g, cnt):
            p0 = g * LANES
            m = xt_v[pl.ds(p0, LANES)] == MASK_TOK
            plsc.store_compressed(
                idx_v.at[pl.ds(cnt, LANES)], p0 + lanes, mask=m)
            return cnt + plsc.all_reduce_population_count(m)[0]

        nmask = lax.fori_loop(0, NGROUPS, scan_body, 0)

        def fire(i, s, sem):
            p = idx_v[pl.ds(i, LANES)][0]
            pltpu.async_copy(
                ls_hbm.at[bi, loff + p], row_v.at[s, pl.ds(0, V)], sem)

        def drain(s, sem):
            pltpu.make_async_copy(
                ls_hbm.at[bi, 0], row_v.at[s, pl.ds(0, V)], sem).wait()

        def reduce_row(i, s, carry):
            pos_acc, neg_acc = carry
            p = idx_v[pl.ds(i, LANES)][0]
            x0r = x0_v[pl.ds(p, LANES)][0]
            negv = row_v[s, pl.ds(x0r, LANES)][0]

            def col_body(j, accs):
                c = j * (LANES * UNROLL)
                return tuple(
                    accs[u] + jnp.exp(row_v[s, pl.ds(c + u * LANES, LANES)])
                    for u in range(UNROLL))

            accs = lax.fori_loop(0, VG // UNROLL, col_body, (zero16,) * UNROLL)
            pos = (accs[0] + accs[1]) + (accs[2] + accs[3])
            pos = pos - jnp.where(
                last_lane, jnp.exp(row_v[s, pl.ds(V - LANES, LANES)]), 0.0)
            return (pos_acc + pos,
                    neg_acc + jnp.full((LANES,), negv, jnp.float32))

        @pl.when(nmask > 0)
        def _():
            fire(0, 0, semA)

        def pair_body(q, carry):
            i0 = 2 * q
            i1 = i0 + 1
            drain(0, semA)

            @pl.when(i1 < nmask)
            def _():
                fire(i1, 1, semB)

            carry = reduce_row(i0, 0, carry)

            def odd_branch(c):
                drain(1, semB)

                @pl.when(i1 + 1 < nmask)
                def _():
                    fire(i1 + 1, 0, semA)

                return reduce_row(i1, 1, c)

            return lax.cond(i1 < nmask, odd_branch, lambda c: c, carry)

        npairs = (nmask + 1) // 2
        pos_acc, neg_acc = lax.fori_loop(
            0, npairs, pair_body, (zero16, zero16))
        cntf = jnp.full((LANES,), nmask, jnp.int32).astype(jnp.float32)
        fin_v[...] = pos_acc + jnp.where(
            lane_is_b, const_v[...] * cntf - ratio_v[...] * neg_acc, 0.0)
        pltpu.sync_copy(fin_v, out_hbm.at[wid])

    return k(ls3d, xt2d, x02d, ratio_pad, const_pad)


def kernel(log_score, sigma_bar, xt, x0):
    expm1_sb = jnp.where(sigma_bar < 0.5, jnp.expm1(sigma_bar),
                         jnp.exp(sigma_bar) - 1.0)
    ratio = 1.0 / expm1_sb
    const = ratio * (jnp.log(ratio) - 1.0)
    ratio_pad = jnp.zeros((LANES,), jnp.float32).at[:B].set(ratio)
    const_pad = jnp.zeros((LANES,), jnp.float32).at[:B].set(const)
    parts = _sc_partials(log_score, xt, x0, ratio_pad, const_pad)
    return parts.sum()


# speedup vs baseline: 2.7526x; 1.0282x over previous
import functools

import jax
import jax.numpy as jnp
from jax import lax
from jax.experimental import pallas as pl
from jax.experimental.pallas import tpu as pltpu
from jax.experimental.pallas import tpu_sc as plsc

NUM_VOCABS = 1024
B, L, V = 8, 4096, 1024
N = B * L
MASK_TOK = NUM_VOCABS - 1

LANES = 16
NC, NS = 2, 16
NW = NC * NS
CHUNK = N // NW
NGROUPS = CHUNK // LANES
VG = V // LANES
UNROLL = 4
WPB = L // CHUNK


def _sc_partials(ls3d, xt2d, x02d, ratio_pad, const_pad):
    mesh = plsc.VectorSubcoreMesh(core_axis_name="c", subcore_axis_name="s")

    @functools.partial(
        pl.kernel,
        mesh=mesh,
        out_type=jax.ShapeDtypeStruct((NW, LANES), jnp.float32),
        compiler_params=pltpu.CompilerParams(
            needs_layout_passes=False, skip_device_barrier=True),
        scratch_types=[
            pltpu.VMEM((CHUNK + LANES,), jnp.int32),
            pltpu.VMEM((CHUNK + LANES,), jnp.int32),
            pltpu.VMEM((CHUNK + LANES,), jnp.int32),
            pltpu.VMEM((2, V + LANES), jnp.float32),
            pltpu.VMEM((LANES,), jnp.float32),
            pltpu.VMEM((LANES,), jnp.float32),
            pltpu.VMEM((LANES,), jnp.float32),
            pltpu.SemaphoreType.DMA,
            pltpu.SemaphoreType.DMA,
            pltpu.SemaphoreType.DMA,
            pltpu.SemaphoreType.DMA,
        ],
    )
    def k(ls_hbm, xt_hbm, x0_hbm, ratio_hbm, const_hbm, out_hbm,
          xt_v, x0_v, idx_v, row_v, ratio_v, const_v, fin_v,
          sem0, sem1, semA, semB):
        wid = lax.axis_index("s") * NC + lax.axis_index("c")
        bi = wid // WPB
        loff = (wid % WPB) * CHUNK
        HALF = CHUNK // 2
        cp_xt0 = pltpu.async_copy(
            xt_hbm.at[bi, pl.ds(loff, HALF)], xt_v.at[pl.ds(0, HALF)], sem0)
        cp_xt1 = pltpu.async_copy(
            xt_hbm.at[bi, pl.ds(loff + HALF, HALF)],
            xt_v.at[pl.ds(HALF, HALF)], sem1)
        cp_x0 = pltpu.async_copy(
            x0_hbm.at[bi, pl.ds(loff, CHUNK)], x0_v.at[pl.ds(0, CHUNK)], semA)
        cp_ratio = pltpu.async_copy(ratio_hbm, ratio_v, semB)

        lanes = lax.broadcasted_iota(jnp.int32, (LANES,), 0)
        zero16 = jnp.zeros((LANES,), jnp.float32)
        row_v[0, pl.ds(V, LANES)] = zero16
        row_v[1, pl.ds(V, LANES)] = zero16
        last_lane = lanes == (LANES - 1)
        lane_is_b = lanes == bi

        def scan_body(g, cnt):
            p0 = g * LANES
            m = xt_v[pl.ds(p0, LANES)] == MASK_TOK
            plsc.store_compressed(
                idx_v.at[pl.ds(cnt, LANES)], p0 + lanes, mask=m)
            return cnt + plsc.all_reduce_population_count(m)[0]

        cp_xt0.wait()
        cp_const = pltpu.async_copy(const_hbm, const_v, sem0)
        nhalf = lax.fori_loop(0, NGROUPS // 2, scan_body, 0)
        cp_xt1.wait()
        nmask = lax.fori_loop(NGROUPS // 2, NGROUPS, scan_body, nhalf)
        cp_x0.wait()
        cp_ratio.wait()

        def fire(i, s, sem):
            p = idx_v[pl.ds(i, LANES)][0]
            pltpu.async_copy(
                ls_hbm.at[bi, loff + p], row_v.at[s, pl.ds(0, V)], sem)

        def drain(s, sem):
            pltpu.make_async_copy(
                ls_hbm.at[bi, 0], row_v.at[s, pl.ds(0, V)], sem).wait()

        def reduce_row(i, s, carry):
            pos_acc, neg_acc = carry
            p = idx_v[pl.ds(i, LANES)][0]
            x0r = x0_v[pl.ds(p, LANES)][0]
            negv = row_v[s, pl.ds(x0r, LANES)][0]

            def col_body(j, accs):
                c = j * (LANES * UNROLL)
                return tuple(
                    accs[u] + jnp.exp(row_v[s, pl.ds(c + u * LANES, LANES)])
                    for u in range(UNROLL))

            accs = lax.fori_loop(0, VG // UNROLL, col_body, (zero16,) * UNROLL)
            pos = (accs[0] + accs[1]) + (accs[2] + accs[3])
            pos = pos - jnp.where(
                last_lane, jnp.exp(row_v[s, pl.ds(V - LANES, LANES)]), 0.0)
            return (pos_acc + pos,
                    neg_acc + jnp.full((LANES,), negv, jnp.float32))

        @pl.when(nmask > 0)
        def _():
            fire(0, 0, semA)

        def pair_body(q, carry):
            i0 = 2 * q
            i1 = i0 + 1
            drain(0, semA)

            @pl.when(i1 < nmask)
            def _():
                fire(i1, 1, semB)

            carry = reduce_row(i0, 0, carry)

            def odd_branch(c):
                drain(1, semB)

                @pl.when(i1 + 1 < nmask)
                def _():
                    fire(i1 + 1, 0, semA)

                return reduce_row(i1, 1, c)

            return lax.cond(i1 < nmask, odd_branch, lambda c: c, carry)

        npairs = (nmask + 1) // 2
        pos_acc, neg_acc = lax.fori_loop(
            0, npairs, pair_body, (zero16, zero16))
        cp_const.wait()
        cntf = jnp.full((LANES,), nmask, jnp.int32).astype(jnp.float32)
        fin_v[...] = pos_acc + jnp.where(
            lane_is_b, const_v[...] * cntf - ratio_v[...] * neg_acc, 0.0)
        pltpu.sync_copy(fin_v, out_hbm.at[wid])

    return k(ls3d, xt2d, x02d, ratio_pad, const_pad)


def kernel(log_score, sigma_bar, xt, x0):
    expm1_sb = jnp.where(sigma_bar < 0.5, jnp.expm1(sigma_bar),
                         jnp.exp(sigma_bar) - 1.0)
    ratio = 1.0 / expm1_sb
    const = ratio * (jnp.log(ratio) - 1.0)
    ratio_pad = jnp.zeros((LANES,), jnp.float32).at[:B].set(ratio)
    const_pad = jnp.zeros((LANES,), jnp.float32).at[:B].set(const)
    parts = _sc_partials(log_score, xt, x0, ratio_pad, const_pad)
    return parts.sum()


# speedup vs baseline: 2.7747x; 1.0080x over previous
import functools

import jax
import jax.numpy as jnp
from jax import lax
from jax.experimental import pallas as pl
from jax.experimental.pallas import tpu as pltpu
from jax.experimental.pallas import tpu_sc as plsc

NUM_VOCABS = 1024
B, L, V = 8, 4096, 1024
N = B * L
MASK_TOK = NUM_VOCABS - 1

LANES = 16
NC, NS = 2, 16
NW = NC * NS
CHUNK = N // NW
NGROUPS = CHUNK // LANES
VG = V // LANES
UNROLL = 4
WPB = L // CHUNK


def _sc_partials(ls3d, xt2d, x02d, ratio_pad, const_pad):
    mesh = plsc.VectorSubcoreMesh(core_axis_name="c", subcore_axis_name="s")

    @functools.partial(
        pl.kernel,
        mesh=mesh,
        out_type=jax.ShapeDtypeStruct((NW, LANES), jnp.float32),
        compiler_params=pltpu.CompilerParams(
            needs_layout_passes=False, skip_device_barrier=True),
        scratch_types=[
            pltpu.VMEM((CHUNK + LANES,), jnp.int32),
            pltpu.VMEM((CHUNK + LANES,), jnp.int32),
            pltpu.VMEM((CHUNK + LANES,), jnp.int32),
            pltpu.VMEM((2, V + LANES), jnp.float32),
            pltpu.VMEM((LANES,), jnp.float32),
            pltpu.VMEM((LANES,), jnp.float32),
            pltpu.VMEM((LANES,), jnp.float32),
            pltpu.SemaphoreType.DMA,
            pltpu.SemaphoreType.DMA,
            pltpu.SemaphoreType.DMA,
            pltpu.SemaphoreType.DMA,
            pltpu.SemaphoreType.DMA,
        ],
    )
    def k(ls_hbm, xt_hbm, x0_hbm, ratio_hbm, const_hbm, out_hbm,
          xt_v, x0_v, idx_v, row_v, ratio_v, const_v, fin_v,
          sem0, sem1, semA, semB, semC):
        wid = lax.axis_index("s") * NC + lax.axis_index("c")
        bi = wid // WPB
        loff = (wid % WPB) * CHUNK
        HALF = CHUNK // 2
        cp_xt0 = pltpu.async_copy(
            xt_hbm.at[bi, pl.ds(loff, HALF)], xt_v.at[pl.ds(0, HALF)], sem0)
        cp_xt1 = pltpu.async_copy(
            xt_hbm.at[bi, pl.ds(loff + HALF, HALF)],
            xt_v.at[pl.ds(HALF, HALF)], sem1)
        cp_x0 = pltpu.async_copy(
            x0_hbm.at[bi, pl.ds(loff, CHUNK)], x0_v.at[pl.ds(0, CHUNK)], semC)
        cp_ratio = pltpu.async_copy(ratio_hbm, ratio_v, semB)

        lanes = lax.broadcasted_iota(jnp.int32, (LANES,), 0)
        zero16 = jnp.zeros((LANES,), jnp.float32)
        row_v[0, pl.ds(V, LANES)] = zero16
        row_v[1, pl.ds(V, LANES)] = zero16
        last_lane = lanes == (LANES - 1)
        lane_is_b = lanes == bi

        def fire(i, s, sem):
            p = idx_v[pl.ds(i, LANES)][0]
            pltpu.async_copy(
                ls_hbm.at[bi, loff + p], row_v.at[s, pl.ds(0, V)], sem)

        def drain(s, sem):
            pltpu.make_async_copy(
                ls_hbm.at[bi, 0], row_v.at[s, pl.ds(0, V)], sem).wait()

        def scan_body(g2, cnt):
            p0 = g2 * (2 * LANES)
            m0 = xt_v[pl.ds(p0, LANES)] == MASK_TOK
            plsc.store_compressed(
                idx_v.at[pl.ds(cnt, LANES)], p0 + lanes, mask=m0)
            c0 = cnt + plsc.all_reduce_population_count(m0)[0]
            p1 = p0 + LANES
            m1 = xt_v[pl.ds(p1, LANES)] == MASK_TOK
            plsc.store_compressed(
                idx_v.at[pl.ds(c0, LANES)], p1 + lanes, mask=m1)
            return c0 + plsc.all_reduce_population_count(m1)[0]

        cp_xt0.wait()
        cp_const = pltpu.async_copy(const_hbm, const_v, sem0)
        nhalf = lax.fori_loop(0, NGROUPS // 4, scan_body, 0)

        @pl.when(nhalf > 0)
        def _():
            fire(0, 0, semA)

        cp_xt1.wait()
        nmask = lax.fori_loop(NGROUPS // 4, NGROUPS // 2, scan_body, nhalf)
        cp_x0.wait()
        cp_ratio.wait()

        @pl.when((nhalf == 0) & (nmask > 0))
        def _():
            fire(0, 0, semA)

        def reduce_row(i, s, carry):
            pos_acc, neg_acc = carry
            p = idx_v[pl.ds(i, LANES)][0]
            x0r = x0_v[pl.ds(p, LANES)][0]
            negv = row_v[s, pl.ds(x0r, LANES)][0]

            def col_body(j, accs):
                c = j * (LANES * UNROLL)
                return tuple(
                    accs[u] + jnp.exp(row_v[s, pl.ds(c + u * LANES, LANES)])
                    for u in range(UNROLL))

            accs = lax.fori_loop(0, VG // UNROLL, col_body, (zero16,) * UNROLL)
            pos = (accs[0] + accs[1]) + (accs[2] + accs[3])
            pos = pos - jnp.where(
                last_lane, jnp.exp(row_v[s, pl.ds(V - LANES, LANES)]), 0.0)
            return (pos_acc + pos,
                    neg_acc + jnp.full((LANES,), negv, jnp.float32))

        def pair_body(q, carry):
            i0 = 2 * q
            i1 = i0 + 1
            drain(0, semA)

            @pl.when(i1 < nmask)
            def _():
                fire(i1, 1, semB)

            carry = reduce_row(i0, 0, carry)

            def odd_branch(c):
                drain(1, semB)

                @pl.when(i1 + 1 < nmask)
                def _():
                    fire(i1 + 1, 0, semA)

                return reduce_row(i1, 1, c)

            return lax.cond(i1 < nmask, odd_branch, lambda c: c, carry)

        npairs = (nmask + 1) // 2
        pos_acc, neg_acc = lax.fori_loop(
            0, npairs, pair_body, (zero16, zero16))
        cp_const.wait()
        cntf = jnp.full((LANES,), nmask, jnp.int32).astype(jnp.float32)
        fin_v[...] = pos_acc + jnp.where(
            lane_is_b, const_v[...] * cntf - ratio_v[...] * neg_acc, 0.0)
        pltpu.sync_copy(fin_v, out_hbm.at[wid])

    return k(ls3d, xt2d, x02d, ratio_pad, const_pad)


def kernel(log_score, sigma_bar, xt, x0):
    expm1_sb = jnp.where(sigma_bar < 0.5, jnp.expm1(sigma_bar),
                         jnp.exp(sigma_bar) - 1.0)
    ratio = 1.0 / expm1_sb
    const = ratio * (jnp.log(ratio) - 1.0)
    ratio_pad = jnp.zeros((LANES,), jnp.float32).at[:B].set(ratio)
    const_pad = jnp.zeros((LANES,), jnp.float32).at[:B].set(const)
    parts = _sc_partials(log_score, xt, x0, ratio_pad, const_pad)
    return parts.sum()
